# baseline (speedup 1.0000x reference)
import jax
import jax.numpy as jnp
from jax import lax
from jax.experimental import pallas as pl
from jax.experimental.pallas import tpu as pltpu
from jax.experimental.pallas import tpu_sc as plsc

VOCAB = 1000000
BATCH = 4096
SEQ_LEN = 200
DEPTH = 32
NW = 32
BROWS_PER_W = BATCH // NW
GROUP_ROWS = 4
GROUP = GROUP_ROWS * SEQ_LEN
N_GROUPS = BROWS_PER_W // GROUP_ROWS
TCH = 512


def _tc_table(table_t):
    def body(x_ref, o_ref):
        x = x_ref[...]
        ji = lax.broadcasted_iota(jnp.int32, (TCH, TCH // 4), 0)
        ki = lax.broadcasted_iota(jnp.int32, (TCH, TCH // 4), 1)
        for r in range(4):
            sel = (ji == 4 * ki + r).astype(jnp.float32)
            xr = lax.dot_general(x, sel, (((1,), (0,)), ((), ())),
                                 precision=lax.Precision.HIGHEST,
                                 preferred_element_type=jnp.float32)
            o_ref[:, r * 32:(r + 1) * 32] = xr.T

    return pl.pallas_call(
        body,
        out_shape=jax.ShapeDtypeStruct((VOCAB // 4, 128), jnp.float32),
        grid=(pl.cdiv(VOCAB, TCH),),
        in_specs=[pl.BlockSpec((32, TCH), lambda i: (0, i))],
        out_specs=pl.BlockSpec((TCH // 4, 128), lambda i: (i, 0)),
    )(table_t)


def _tc_out(packed3):
    def body(x_ref, o_ref):
        for k in range(50):
            x = x_ref[:, k, :]
            for m in range(4):
                yt = x[:, m * 32:(m + 1) * 32].T
                for dt in range(4):
                    o_ref[4 * k + m, dt, 0, :, :] = yt[dt * 8:(dt + 1) * 8, :]

    return pl.pallas_call(
        body,
        out_shape=jax.ShapeDtypeStruct((SEQ_LEN, 4, NW, 8, BROWS_PER_W),
                                       jnp.float32),
        grid=(NW,),
        in_specs=[pl.BlockSpec((BROWS_PER_W, 50, 128), lambda bt: (bt, 0, 0))],
        out_specs=pl.BlockSpec((SEQ_LEN, 4, 1, 8, BROWS_PER_W),
                               lambda bt: (0, 0, bt, 0, 0)),
    )(packed3)


def _sc_body(seq_hbm, pos_hbm, table_hbm, out_hbm,
             idx_v, rows0, rows1, pos_v, gsem0, gsem1, osem0, osem1):
    wid = lax.axis_index("s") * 2 + lax.axis_index("c")
    brow0 = wid * BROWS_PER_W
    pltpu.sync_copy(seq_hbm.at[pl.ds(brow0, BROWS_PER_W)], idx_v)
    pltpu.sync_copy(pos_hbm, pos_v)

    def gather(g, rows_ref, sem):
        for j in range(GROUP_ROWS):
            pltpu.async_copy(table_hbm.at[idx_v.at[g * GROUP_ROWS + j]],
                             rows_ref.at[pl.ds(j * SEQ_LEN, SEQ_LEN)], sem)

    def gather_wait(rows_ref, sem):
        for j in range(GROUP_ROWS):
            pltpu.make_async_copy(table_hbm.at[idx_v.at[0]],
                                  rows_ref.at[pl.ds(j * SEQ_LEN, SEQ_LEN)],
                                  sem).wait()

    def store(g, rows_ref, sem):
        pltpu.async_copy(
            rows_ref,
            out_hbm.at[pl.ds((brow0 + g * GROUP_ROWS) * SEQ_LEN, GROUP)], sem)

    def store_wait(rows_ref, sem):
        pltpu.make_async_copy(rows_ref,
                              out_hbm.at[pl.ds(brow0 * SEQ_LEN, GROUP)],
                              sem).wait()

    def add_pos(rows_ref):
        @pl.loop(0, SEQ_LEN, unroll=2)
        def _(l):
            p0 = pos_v[l, pl.ds(0, 16)]
            p1 = pos_v[l, pl.ds(16, 16)]
            for j in range(GROUP_ROWS):
                plsc.addupdate(rows_ref.at[j * SEQ_LEN + l, pl.ds(0, 16)], p0)
                plsc.addupdate(rows_ref.at[j * SEQ_LEN + l, pl.ds(16, 16)], p1)

    gather(0, rows0, gsem0)

    def pipe_body(i, carry):
        g0 = 2 * i

        @pl.when(g0 > 0)
        def _():
            store_wait(rows1, osem1)
        gather(g0 + 1, rows1, gsem1)

        gather_wait(rows0, gsem0)
        add_pos(rows0)
        store(g0, rows0, osem0)

        @pl.when(g0 + 2 < N_GROUPS)
        def _():
            store_wait(rows0, osem0)
            gather(g0 + 2, rows0, gsem0)

        gather_wait(rows1, gsem1)
        add_pos(rows1)
        store(g0 + 1, rows1, osem1)
        return carry

    lax.fori_loop(0, N_GROUPS // 2, pipe_body, 0)
    store_wait(rows0, osem0)
    store_wait(rows1, osem1)


def _sc_gather(seq, pos_table, tablin):
    mesh = plsc.VectorSubcoreMesh(core_axis_name="c", subcore_axis_name="s")
    return pl.kernel(
        _sc_body,
        out_type=jax.ShapeDtypeStruct((BATCH * SEQ_LEN, DEPTH), jnp.float32),
        mesh=mesh,
        compiler_params=pltpu.CompilerParams(use_tc_tiling_on_sc=False),
        scratch_types=[
            pltpu.VMEM((BROWS_PER_W, SEQ_LEN), jnp.int32),
            pltpu.VMEM((GROUP, DEPTH), jnp.float32),
            pltpu.VMEM((GROUP, DEPTH), jnp.float32),
            pltpu.VMEM((SEQ_LEN, DEPTH), jnp.float32),
            pltpu.SemaphoreType.DMA,
            pltpu.SemaphoreType.DMA,
            pltpu.SemaphoreType.DMA,
            pltpu.SemaphoreType.DMA,
        ],
    )(seq, pos_table, tablin)


def kernel(seq, token_table, pos_table):
    packed = _tc_table(token_table.T)
    tablin = packed.reshape(VOCAB, DEPTH)
    emb = _sc_gather(seq.astype(jnp.int32), pos_table, tablin)
    out5 = _tc_out(emb.reshape(BATCH, 50, 128))
    return out5.transpose(2, 4, 0, 1, 3).reshape(BATCH, SEQ_LEN, DEPTH)


# speedup vs baseline: 1.2051x; 1.2051x over previous
import jax
import jax.numpy as jnp
from jax import lax
from jax.experimental import pallas as pl
from jax.experimental.pallas import tpu as pltpu
from jax.experimental.pallas import tpu_sc as plsc

VOCAB = 1000000
BATCH = 4096
SEQ_LEN = 200
DEPTH = 32
NW = 32
BROWS_PER_W = BATCH // NW
TCH = 512


def _tc_pad(table_t):
    def body(x_ref, o_ref):
        o_ref[:, 0:DEPTH] = x_ref[...].T

    return pl.pallas_call(
        body,
        out_shape=jax.ShapeDtypeStruct((VOCAB, 128), jnp.float32),
        grid=(pl.cdiv(VOCAB, TCH),),
        in_specs=[pl.BlockSpec((DEPTH, TCH), lambda i: (0, i))],
        out_specs=pl.BlockSpec((TCH, 128), lambda i: (i, 0)),
    )(table_t)


def _tc_out(packed3):
    def body(x_ref, o_ref):
        for k in range(50):
            x = x_ref[:, k, :]
            for m in range(4):
                yt = x[:, m * 32:(m + 1) * 32].T
                for dt in range(4):
                    o_ref[4 * k + m, dt, 0, :, :] = yt[dt * 8:(dt + 1) * 8, :]

    return pl.pallas_call(
        body,
        out_shape=jax.ShapeDtypeStruct((SEQ_LEN, 4, NW, 8, BROWS_PER_W),
                                       jnp.float32),
        grid=(NW,),
        in_specs=[pl.BlockSpec((BROWS_PER_W, 56, 128), lambda bt: (bt, 0, 0))],
        out_specs=pl.BlockSpec((SEQ_LEN, 4, 1, 8, BROWS_PER_W),
                               lambda bt: (0, 0, bt, 0, 0)),
    )(packed3)


def _sc_body(seq_hbm, pos_hbm, table_hbm, out_hbm,
             idx_v, rows0, rows1, pos_v, gsem0, gsem1, osem0, osem1):
    wid = lax.axis_index("s") * 2 + lax.axis_index("c")
    brow0 = wid * BROWS_PER_W
    pltpu.sync_copy(seq_hbm.at[pl.ds(brow0, BROWS_PER_W)], idx_v)
    pltpu.sync_copy(pos_hbm, pos_v)

    def gather(g, rows_ref, sem):
        pltpu.async_copy(table_hbm.at[idx_v.at[g]], rows_ref, sem)

    def gather_wait(rows_ref, sem):
        pltpu.make_async_copy(table_hbm.at[idx_v.at[0]], rows_ref, sem).wait()

    def store(g, rows_ref, sem):
        pltpu.async_copy(rows_ref.at[:, pl.ds(0, DEPTH)],
                         out_hbm.at[pl.ds((brow0 + g) * 224, SEQ_LEN)],
                         sem)

    def store_wait(rows_ref, sem):
        pltpu.make_async_copy(rows_ref.at[:, pl.ds(0, DEPTH)],
                              out_hbm.at[pl.ds(brow0 * 224, SEQ_LEN)],
                              sem).wait()

    def add_pos(rows_ref):
        @pl.loop(0, SEQ_LEN, unroll=4)
        def _(l):
            plsc.addupdate(rows_ref.at[l, pl.ds(0, 16)],
                           pos_v[l, pl.ds(0, 16)])
            plsc.addupdate(rows_ref.at[l, pl.ds(16, 16)],
                           pos_v[l, pl.ds(16, 16)])

    gather(0, rows0, gsem0)

    def pipe_body(i, carry):
        g0 = 2 * i

        @pl.when(g0 > 0)
        def _():
            store_wait(rows1, osem1)
        gather(g0 + 1, rows1, gsem1)

        gather_wait(rows0, gsem0)
        add_pos(rows0)
        store(g0, rows0, osem0)

        @pl.when(g0 + 2 < BROWS_PER_W)
        def _():
            store_wait(rows0, osem0)
            gather(g0 + 2, rows0, gsem0)

        gather_wait(rows1, gsem1)
        add_pos(rows1)
        store(g0 + 1, rows1, osem1)
        return carry

    lax.fori_loop(0, BROWS_PER_W // 2, pipe_body, 0)
    store_wait(rows0, osem0)
    store_wait(rows1, osem1)


def _sc_gather(seq, pos_table, tabpad):
    mesh = plsc.VectorSubcoreMesh(core_axis_name="c", subcore_axis_name="s")
    return pl.kernel(
        _sc_body,
        out_type=jax.ShapeDtypeStruct((BATCH * 224, DEPTH), jnp.float32),
        mesh=mesh,
        compiler_params=pltpu.CompilerParams(use_tc_tiling_on_sc=False),
        scratch_types=[
            pltpu.VMEM((BROWS_PER_W, SEQ_LEN), jnp.int32),
            pltpu.VMEM((SEQ_LEN, 128), jnp.float32),
            pltpu.VMEM((SEQ_LEN, 128), jnp.float32),
            pltpu.VMEM((SEQ_LEN, DEPTH), jnp.float32),
            pltpu.SemaphoreType.DMA,
            pltpu.SemaphoreType.DMA,
            pltpu.SemaphoreType.DMA,
            pltpu.SemaphoreType.DMA,
        ],
    )(seq, pos_table, tabpad)


def kernel(seq, token_table, pos_table):
    tabpad = _tc_pad(token_table.T)
    emb = _sc_gather(seq.astype(jnp.int32), pos_table, tabpad)
    out5 = _tc_out(emb.reshape(BATCH, 56, 128))
    return out5.transpose(2, 4, 0, 1, 3).reshape(BATCH, SEQ_LEN, DEPTH)


# speedup vs baseline: 2.5024x; 2.0766x over previous
import jax
import jax.numpy as jnp
from jax import lax
from jax.experimental import pallas as pl
from jax.experimental.pallas import tpu as pltpu
from jax.experimental.pallas import tpu_sc as plsc

VOCAB = 1000000
BATCH = 4096
SEQ_LEN = 200
DEPTH = 32
NW = 32
BROWS_PER_W = BATCH // NW
TCH = 512


def _tc_pad(table_t):
    def body(x_ref, o_ref):
        o_ref[:, 0:DEPTH] = x_ref[...].T

    return pl.pallas_call(
        body,
        out_shape=jax.ShapeDtypeStruct((VOCAB, 128), jnp.float32),
        grid=(pl.cdiv(VOCAB, TCH),),
        in_specs=[pl.BlockSpec((DEPTH, TCH), lambda i: (0, i))],
        out_specs=pl.BlockSpec((TCH, 128), lambda i: (i, 0)),
    )(table_t)


def _tc_out(packed3):
    def body(x_ref, o_ref):
        for k in range(50):
            x = x_ref[:, k, :]
            for m in range(4):
                yt = x[:, m * 32:(m + 1) * 32].T
                for dt in range(4):
                    o_ref[4 * k + m, dt, 0, :, :] = yt[dt * 8:(dt + 1) * 8, :]

    return pl.pallas_call(
        body,
        out_shape=jax.ShapeDtypeStruct((SEQ_LEN, 4, NW, 8, BROWS_PER_W),
                                       jnp.float32),
        grid=(NW,),
        in_specs=[pl.BlockSpec((BROWS_PER_W, 56, 128), lambda bt: (bt, 0, 0))],
        out_specs=pl.BlockSpec((SEQ_LEN, 4, 1, 8, BROWS_PER_W),
                               lambda bt: (0, 0, bt, 0, 0)),
    )(packed3)


def _sc_body(seq_hbm, pos_hbm, table_hbm, out_hbm,
             idx_v, rows0, rows1, pos_v, gsem0, gsem1, osem0, osem1):
    wid = lax.axis_index("s") * 2 + lax.axis_index("c")
    brow0 = wid * BROWS_PER_W
    pltpu.sync_copy(seq_hbm.at[pl.ds(brow0, BROWS_PER_W)], idx_v)
    pltpu.sync_copy(pos_hbm, pos_v)

    def gather(g, rows_ref, sem):
        pltpu.async_copy(table_hbm.at[idx_v.at[g]], rows_ref, sem)

    def gather_wait(rows_ref, sem):
        pltpu.make_async_copy(table_hbm.at[idx_v.at[0]], rows_ref, sem).wait()

    def store(g, rows_ref, sem):
        pltpu.async_copy(rows_ref,
                         out_hbm.at[pl.ds((brow0 + g) * 224, SEQ_LEN)],
                         sem)

    def store_wait(rows_ref, sem):
        pltpu.make_async_copy(rows_ref,
                              out_hbm.at[pl.ds(brow0 * 224, SEQ_LEN)],
                              sem).wait()

    def add_pos(rows_ref):
        @pl.loop(0, SEQ_LEN, unroll=4)
        def _(l):
            plsc.addupdate(rows_ref.at[l, pl.ds(0, 16)],
                           pos_v[l, pl.ds(0, 16)])
            plsc.addupdate(rows_ref.at[l, pl.ds(16, 16)],
                           pos_v[l, pl.ds(16, 16)])

    gather(0, rows0, gsem0)

    def pipe_body(i, carry):
        g0 = 2 * i

        @pl.when(g0 > 0)
        def _():
            store_wait(rows1, osem1)
        gather(g0 + 1, rows1, gsem1)

        gather_wait(rows0, gsem0)
        add_pos(rows0)
        store(g0, rows0, osem0)

        @pl.when(g0 + 2 < BROWS_PER_W)
        def _():
            store_wait(rows0, osem0)
            gather(g0 + 2, rows0, gsem0)

        gather_wait(rows1, gsem1)
        add_pos(rows1)
        store(g0 + 1, rows1, osem1)
        return carry

    lax.fori_loop(0, BROWS_PER_W // 2, pipe_body, 0)
    store_wait(rows0, osem0)
    store_wait(rows1, osem1)


def _sc_gather(seq, pos_table, tabpad):
    mesh = plsc.VectorSubcoreMesh(core_axis_name="c", subcore_axis_name="s")
    return pl.kernel(
        _sc_body,
        out_type=jax.ShapeDtypeStruct((BATCH * 224, DEPTH), jnp.float32),
        mesh=mesh,
        compiler_params=pltpu.CompilerParams(use_tc_tiling_on_sc=False),
        scratch_types=[
            pltpu.VMEM((BROWS_PER_W, SEQ_LEN), jnp.int32),
            pltpu.VMEM((SEQ_LEN, DEPTH), jnp.float32),
            pltpu.VMEM((SEQ_LEN, DEPTH), jnp.float32),
            pltpu.VMEM((SEQ_LEN, DEPTH), jnp.float32),
            pltpu.SemaphoreType.DMA,
            pltpu.SemaphoreType.DMA,
            pltpu.SemaphoreType.DMA,
            pltpu.SemaphoreType.DMA,
        ],
    )(seq, pos_table, tabpad)


def kernel(seq, token_table, pos_table):
    emb = _sc_gather(seq.astype(jnp.int32), pos_table, token_table)
    out5 = _tc_out(emb.reshape(BATCH, 56, 128))
    return out5.transpose(2, 4, 0, 1, 3).reshape(BATCH, SEQ_LEN, DEPTH)


# speedup vs baseline: 2.5051x; 1.0010x over previous
import jax
import jax.numpy as jnp
from jax import lax
from jax.experimental import pallas as pl
from jax.experimental.pallas import tpu as pltpu
from jax.experimental.pallas import tpu_sc as plsc

VOCAB = 1000000
BATCH = 4096
SEQ_LEN = 200
DEPTH = 32
NW = 32
BROWS_PER_W = BATCH // NW


def _tc_out(packed3):
    def body(x_ref, o_ref):
        for k in range(50):
            x = x_ref[:, k, :]
            for m in range(4):
                yt = x[:, m * 32:(m + 1) * 32].T
                for dt in range(4):
                    o_ref[4 * k + m, dt, 0, :, :] = yt[dt * 8:(dt + 1) * 8, :]

    return pl.pallas_call(
        body,
        out_shape=jax.ShapeDtypeStruct((SEQ_LEN, 4, NW, 8, BROWS_PER_W),
                                       jnp.float32),
        grid=(NW,),
        in_specs=[pl.BlockSpec((BROWS_PER_W, 56, 128), lambda bt: (bt, 0, 0))],
        out_specs=pl.BlockSpec((SEQ_LEN, 4, 1, 8, BROWS_PER_W),
                               lambda bt: (0, 0, bt, 0, 0)),
    )(packed3)


def _sc_body(seq_hbm, pos_hbm, table_hbm, out_hbm,
             idx_v, rows0, rows1, pos_v, gsem0, gsem1, osem0, osem1):
    wid = lax.axis_index("s") * 2 + lax.axis_index("c")
    brow0 = wid * BROWS_PER_W
    pltpu.sync_copy(seq_hbm.at[pl.ds(brow0, BROWS_PER_W)], idx_v)
    pltpu.sync_copy(pos_hbm, pos_v)

    def gather(g, rows_ref, sem):
        pltpu.async_copy(table_hbm.at[idx_v.at[g]], rows_ref, sem)

    def gather_wait(rows_ref, sem):
        pltpu.make_async_copy(table_hbm.at[idx_v.at[0]], rows_ref, sem).wait()

    def store(g, rows_ref, sem):
        pltpu.async_copy(rows_ref,
                         out_hbm.at[pl.ds((brow0 + g) * 224, SEQ_LEN)],
                         sem)

    def store_wait(rows_ref, sem):
        pltpu.make_async_copy(rows_ref,
                              out_hbm.at[pl.ds(brow0 * 224, SEQ_LEN)],
                              sem).wait()

    def add_pos(rows_ref):
        @pl.loop(0, SEQ_LEN, unroll=4)
        def _(l):
            plsc.addupdate(rows_ref.at[l, pl.ds(0, 16)],
                           pos_v[l, pl.ds(0, 16)])
            plsc.addupdate(rows_ref.at[l, pl.ds(16, 16)],
                           pos_v[l, pl.ds(16, 16)])

    gather(0, rows0, gsem0)

    def pipe_body(i, carry):
        g0 = 2 * i

        @pl.when(g0 > 0)
        def _():
            store_wait(rows1, osem1)
        gather(g0 + 1, rows1, gsem1)

        gather_wait(rows0, gsem0)
        add_pos(rows0)
        store(g0, rows0, osem0)

        @pl.when(g0 + 2 < BROWS_PER_W)
        def _():
            store_wait(rows0, osem0)
            gather(g0 + 2, rows0, gsem0)

        gather_wait(rows1, gsem1)
        add_pos(rows1)
        store(g0 + 1, rows1, osem1)
        return carry

    lax.fori_loop(0, BROWS_PER_W // 2, pipe_body, 0)
    store_wait(rows0, osem0)
    store_wait(rows1, osem1)


def _sc_gather(seq, pos_table, tabpad):
    mesh = plsc.VectorSubcoreMesh(core_axis_name="c", subcore_axis_name="s")
    return pl.kernel(
        _sc_body,
        out_type=jax.ShapeDtypeStruct((BATCH * 224, DEPTH), jnp.float32),
        mesh=mesh,
        compiler_params=pltpu.CompilerParams(use_tc_tiling_on_sc=False),
        scratch_types=[
            pltpu.VMEM((BROWS_PER_W, SEQ_LEN), jnp.int32),
            pltpu.VMEM((SEQ_LEN, DEPTH), jnp.float32),
            pltpu.VMEM((SEQ_LEN, DEPTH), jnp.float32),
            pltpu.VMEM((SEQ_LEN, DEPTH), jnp.float32),
            pltpu.SemaphoreType.DMA,
            pltpu.SemaphoreType.DMA,
            pltpu.SemaphoreType.DMA,
            pltpu.SemaphoreType.DMA,
        ],
    )(seq, pos_table, tabpad)


def kernel(seq, token_table, pos_table):
    emb = _sc_gather(seq.astype(jnp.int32), pos_table, token_table)
    out5 = _tc_out(emb.reshape(BATCH, 56, 128))
    return out5.transpose(2, 4, 0, 1, 3).reshape(BATCH, SEQ_LEN, DEPTH)
